# baseline (speedup 1.0000x reference)
import functools

import jax
import jax.numpy as jnp
from jax import lax
from jax.experimental import pallas as pl
from jax.experimental.pallas import tpu as pltpu
from jax.experimental.pallas import tpu_sc as plsc

N = 10000
E = 320000
H = 128

NODE_BLK = 1000
EDGE_BLK = 2000

NTILES = 32
NPT = 320
NPAD = NTILES * NPT
TRASH = NPT
C = 6400
GB = 96
LCAP = C + GB + 32
DL_SHIFT = 19


def _node_body(nf_ref, Wr1_ref, br1_ref, Wr2_ref, br2_ref, W1_ref, b1_ref,
               x_ref, x1_ref):
    nf = nf_ref[...]
    t = jnp.dot(nf, Wr1_ref[...], preferred_element_type=jnp.float32) + br1_ref[...]
    x = jnp.dot(t, Wr2_ref[...], preferred_element_type=jnp.float32) + br2_ref[...] + nf
    x_ref[...] = x
    x1_ref[...] = jnp.dot(x, W1_ref[...], preferred_element_type=jnp.float32) + b1_ref[...]


def _edge_body(es_ref, ed_ref, Wm1t_ref, Wm1b_ref, bm1_ref, Wf_ref, bf_ref,
               wk_ref, bk_ref, ef_ref):
    z = (jnp.dot(es_ref[...].astype(jnp.bfloat16), Wm1t_ref[...],
                 preferred_element_type=jnp.float32)
         + jnp.dot(ed_ref[...].astype(jnp.bfloat16), Wm1b_ref[...],
                   preferred_element_type=jnp.float32)
         + bm1_ref[...])
    zl = jnp.maximum(z, 0.2 * z)
    f = jnp.dot(zl.astype(jnp.bfloat16), Wf_ref[...],
                preferred_element_type=jnp.float32) + bf_ref[...]
    k_log = jnp.sum(zl * wk_ref[...], axis=1, keepdims=True) + bk_ref[...]
    ef_ref[...] = f * jax.nn.sigmoid(k_log)


def _reduce_body(x_ref, x1_ref, nf_ref, Wrx_ref, Wr1n_ref, Wr2n_ref,
                 bred_ref, W2_ref, b2_ref, out_ref):
    n1 = nf_ref[:, :H]
    n2 = nf_ref[:, H:]
    n2 = jnp.where(jnp.isneginf(n2), 0.0, n2)
    new_x = (jnp.dot(x_ref[...], Wrx_ref[...], preferred_element_type=jnp.float32)
             + jnp.dot(n1, Wr1n_ref[...], preferred_element_type=jnp.float32)
             + jnp.dot(n2, Wr2n_ref[...], preferred_element_type=jnp.float32)
             + bred_ref[...])
    x2 = jnp.dot(new_x, W2_ref[...], preferred_element_type=jnp.float32) + b2_ref[...]
    out_ref[...] = x1_ref[...] + x2


def _seg_body(dst_hbm, ef_hbm, out_hbm, acc, dstbuf, dstbuf2, listbuf, rows,
              idxbuf, dlbuf, sem, sem_a, sem_b):
    wid = lax.axis_index("s") * 2 + lax.axis_index("c")
    lo = wid * NPT
    iota = lax.iota(jnp.int32, 16)
    zeros16 = jnp.zeros((16,), jnp.float32)
    neginf16 = jnp.full((16,), -jnp.inf, jnp.float32)

    def init_row(r, carry):
        for j in range(8):
            acc[r, pl.ds(16 * j, 16)] = zeros16
        for j in range(8, 16):
            acc[r, pl.ds(16 * j, 16)] = neginf16
        return carry

    lax.fori_loop(0, NPT + 1, init_row, 0)

    def process_batch(b, carry):
        base = b * GB
        for t in range(GB // 16):
            pk = listbuf[pl.ds(base + 16 * t, 16)]
            idxbuf[pl.ds(16 * t, 16)] = pk & ((1 << DL_SHIFT) - 1)
            dlbuf[pl.ds(16 * t, 16)] = lax.shift_right_logical(pk, DL_SHIFT)
        pltpu.async_copy(ef_hbm.at[idxbuf], rows, sem).wait()

        def acc_group(g, c2):
            dlv = dlbuf[pl.ds(16 * g, 16)]
            for j in range(16):
                node = dlv[j]
                i = 16 * g + j
                for t in range(8):
                    sl = pl.ds(16 * t, 16)
                    plsc.addupdate(acc.at[node, sl], rows[i, sl])
                for t in range(8, 16):
                    sl = pl.ds(16 * t, 16)
                    acc[node, sl] = jnp.maximum(acc[node, sl], rows[i, sl])
            return c2

        lax.fori_loop(0, GB // 16, acc_group, 0)
        return carry

    def drain_full(off):
        nb = off // GB
        lax.fori_loop(0, nb, process_batch, 0)
        for t in range(GB // 16):
            listbuf[pl.ds(16 * t, 16)] = listbuf[pl.ds(nb * GB + 16 * t, 16)]
        return off - nb * GB

    def scan_buf(buf, ci, off):
        def scan_group(g, off):
            for t in range(4):
                v = 4 * g + t
                ids = buf[pl.ds(16 * v, 16)]
                dl = ids - lo
                m = (dl >= 0) & (dl < NPT)
                pc = plsc.all_reduce_population_count(m)[0]
                cs = plsc.cumsum(m.astype(jnp.int32))
                pos = jnp.where(m, off + cs - 1, LCAP - 1)
                eids = (ci * C + 16 * v) + iota
                pk = eids | (dl << DL_SHIFT)
                plsc.store_scatter(listbuf, [pos], pk)
                off = off + pc
            return off

        return lax.fori_loop(0, C // 64, scan_group, off)

    NCHUNK = E // C

    def start_load(buf, s, ci):
        pltpu.async_copy(dst_hbm.at[pl.ds(ci * C, C)], buf, s)

    def wait_load(buf, s):
        pltpu.make_async_copy(dst_hbm.at[pl.ds(0, C)], buf, s).wait()

    start_load(dstbuf, sem_a, 0)
    start_load(dstbuf2, sem_b, 1)

    def chunk_pair(i, off):
        ca = 2 * i
        wait_load(dstbuf, sem_a)
        off = scan_buf(dstbuf, ca, off)
        start_load(dstbuf, sem_a, jnp.where(ca + 2 < NCHUNK, ca + 2, 0))
        off = drain_full(off)
        wait_load(dstbuf2, sem_b)
        off = scan_buf(dstbuf2, ca + 1, off)
        start_load(dstbuf2, sem_b, jnp.where(ca + 3 < NCHUNK, ca + 3, 0))
        off = drain_full(off)
        return off

    off = lax.fori_loop(0, NCHUNK // 2, chunk_pair, jnp.int32(0))
    wait_load(dstbuf, sem_a)
    wait_load(dstbuf2, sem_b)

    pad = jnp.full((16,), TRASH << DL_SHIFT, jnp.int32)
    def pad_tail(t, carry):
        listbuf[pl.ds(off + 16 * t, 16)] = pad
        return carry
    lax.fori_loop(0, GB // 16, pad_tail, 0)
    off_padded = ((off + GB - 1) // GB) * GB
    drain_full(off_padded)

    pltpu.sync_copy(acc.at[pl.ds(0, NPT)], out_hbm.at[pl.ds(wid * NPT, NPT)])


EPT = E // NTILES
GCH = 80


def _gather_body(src_hbm, dst_hbm, x_hbm, es_hbm, ed_hbm,
                 sidx, didx, srows, drows, sem1, sem2):
    wid = lax.axis_index("s") * 2 + lax.axis_index("c")
    base = wid * EPT
    pltpu.sync_copy(src_hbm.at[pl.ds(base, EPT)], sidx)
    pltpu.sync_copy(dst_hbm.at[pl.ds(base, EPT)], didx)

    def chunk(c, carry):
        o = c * GCH
        cp1 = pltpu.async_copy(x_hbm.at[sidx.at[pl.ds(o, GCH)]], srows, sem1)
        cp2 = pltpu.async_copy(x_hbm.at[didx.at[pl.ds(o, GCH)]], drows, sem2)
        cp1.wait()
        cp2.wait()
        pltpu.sync_copy(srows, es_hbm.at[pl.ds(base + o, GCH)])
        pltpu.sync_copy(drows, ed_hbm.at[pl.ds(base + o, GCH)])
        return carry

    lax.fori_loop(0, EPT // GCH, chunk, 0)


_edge_gather = functools.partial(
    pl.kernel,
    out_type=(jax.ShapeDtypeStruct((E, H), jnp.float32),
              jax.ShapeDtypeStruct((E, H), jnp.float32)),
    mesh=plsc.VectorSubcoreMesh(core_axis_name="c", subcore_axis_name="s",
                                num_cores=2, num_subcores=16),
    scratch_types=[
        pltpu.VMEM((EPT,), jnp.int32),
        pltpu.VMEM((EPT,), jnp.int32),
        pltpu.VMEM((GCH, H), jnp.float32),
        pltpu.VMEM((GCH, H), jnp.float32),
        pltpu.SemaphoreType.DMA,
        pltpu.SemaphoreType.DMA,
    ],
    compiler_params=pltpu.CompilerParams(needs_layout_passes=False),
)(_gather_body)


_segment_reduce = functools.partial(
    pl.kernel,
    out_type=jax.ShapeDtypeStruct((NPAD, 2 * H), jnp.float32),
    mesh=plsc.VectorSubcoreMesh(core_axis_name="c", subcore_axis_name="s",
                                num_cores=2, num_subcores=16),
    scratch_types=[
        pltpu.VMEM((NPT + 1, 2 * H), jnp.float32),
        pltpu.VMEM((C,), jnp.int32),
        pltpu.VMEM((C,), jnp.int32),
        pltpu.VMEM((LCAP,), jnp.int32),
        pltpu.VMEM((GB, 2 * H), jnp.float32),
        pltpu.VMEM((GB,), jnp.int32),
        pltpu.VMEM((GB,), jnp.int32),
        pltpu.SemaphoreType.DMA,
        pltpu.SemaphoreType.DMA,
        pltpu.SemaphoreType.DMA,
    ],
    compiler_params=pltpu.CompilerParams(needs_layout_passes=False),
)(_seg_body)


def _full(shape):
    return pl.BlockSpec(shape, lambda i: (0,) * len(shape))


def kernel(nf_gc, edge_index, Wr1, br1, Wr2, br2, W1, b1, W2, b2,
           Wm1, bm1, Wm2, bm2, Wred, bred):
    n, h = nf_gc.shape
    src = edge_index[0]
    dst = edge_index[1].astype(jnp.int32)

    Wm1t = Wm1[:h]
    Wm1b = Wm1[h:]
    wk = Wm2[:, :1].T
    bk = bm2[:1].reshape(1, 1)
    Wf = Wm2[:, 1:]
    bf = bm2[1:].reshape(1, 2 * h)

    grid_n = n // NODE_BLK
    x, x1 = pl.pallas_call(
        _node_body,
        grid=(grid_n,),
        in_specs=[
            pl.BlockSpec((NODE_BLK, h), lambda i: (i, 0)),
            _full((h, h)), _full((1, h)), _full((h, h)), _full((1, h)),
            _full((h, h)), _full((1, h)),
        ],
        out_specs=[
            pl.BlockSpec((NODE_BLK, h), lambda i: (i, 0)),
            pl.BlockSpec((NODE_BLK, h), lambda i: (i, 0)),
        ],
        out_shape=[
            jax.ShapeDtypeStruct((n, h), jnp.float32),
            jax.ShapeDtypeStruct((n, h), jnp.float32),
        ],
    )(nf_gc, Wr1, br1.reshape(1, h), Wr2, br2.reshape(1, h),
      W1, b1.reshape(1, h))

    es, ed = _edge_gather(src.astype(jnp.int32), dst, x)

    grid_e = E // EDGE_BLK
    ef = pl.pallas_call(
        _edge_body,
        grid=(grid_e,),
        in_specs=[
            pl.BlockSpec((EDGE_BLK, h), lambda i: (i, 0)),
            pl.BlockSpec((EDGE_BLK, h), lambda i: (i, 0)),
            _full((h, 2 * h)), _full((h, 2 * h)),
            _full((1, 2 * h)), _full((2 * h, 2 * h)), _full((1, 2 * h)),
            _full((1, 2 * h)), _full((1, 1)),
        ],
        out_specs=pl.BlockSpec((EDGE_BLK, 2 * h), lambda i: (i, 0)),
        out_shape=jax.ShapeDtypeStruct((E, 2 * h), jnp.float32),
    )(es, ed, Wm1t.astype(jnp.bfloat16), Wm1b.astype(jnp.bfloat16),
      bm1.reshape(1, 2 * h), Wf.astype(jnp.bfloat16), bf, wk, bk)

    nfoc = _segment_reduce(dst, ef)[:n]

    out = pl.pallas_call(
        _reduce_body,
        grid=(grid_n,),
        in_specs=[
            pl.BlockSpec((NODE_BLK, h), lambda i: (i, 0)),
            pl.BlockSpec((NODE_BLK, h), lambda i: (i, 0)),
            pl.BlockSpec((NODE_BLK, 2 * h), lambda i: (i, 0)),
            _full((h, h)), _full((h, h)), _full((h, h)), _full((1, h)),
            _full((h, h)), _full((1, h)),
        ],
        out_specs=pl.BlockSpec((NODE_BLK, h), lambda i: (i, 0)),
        out_shape=jax.ShapeDtypeStruct((n, h), jnp.float32),
    )(x, x1, nfoc, Wred[:h], Wred[h:2 * h], Wred[2 * h:],
      bred.reshape(1, h), W2, b2.reshape(1, h))

    return out


# speedup vs baseline: 1.0487x; 1.0487x over previous
import functools

import jax
import jax.numpy as jnp
from jax import lax
from jax.experimental import pallas as pl
from jax.experimental.pallas import tpu as pltpu
from jax.experimental.pallas import tpu_sc as plsc

N = 10000
E = 320000
H = 128

NODE_BLK = 1000
EDGE_BLK = 2000

NTILES = 32
NPT = 320
NPAD = NTILES * NPT
TRASH = NPT
C = 3200
GB = 128
LCAP = C + GB + 32
DL_SHIFT = 19


def _node_body(nf_ref, Wr1_ref, br1_ref, Wr2_ref, br2_ref, W1_ref, b1_ref,
               x_ref, x1_ref):
    nf = nf_ref[...]
    t = jnp.dot(nf, Wr1_ref[...], preferred_element_type=jnp.float32) + br1_ref[...]
    x = jnp.dot(t, Wr2_ref[...], preferred_element_type=jnp.float32) + br2_ref[...] + nf
    x_ref[...] = x
    x1_ref[...] = jnp.dot(x, W1_ref[...], preferred_element_type=jnp.float32) + b1_ref[...]


def _edge_body(es_ref, ed_ref, Wm1t_ref, Wm1b_ref, bm1_ref, Wf_ref, bf_ref,
               wk_ref, bk_ref, ef_ref):
    z = (jnp.dot(es_ref[...].astype(jnp.bfloat16), Wm1t_ref[...],
                 preferred_element_type=jnp.float32)
         + jnp.dot(ed_ref[...].astype(jnp.bfloat16), Wm1b_ref[...],
                   preferred_element_type=jnp.float32)
         + bm1_ref[...])
    zl = jnp.maximum(z, 0.2 * z)
    f = jnp.dot(zl.astype(jnp.bfloat16), Wf_ref[...],
                preferred_element_type=jnp.float32) + bf_ref[...]
    k_log = jnp.sum(zl * wk_ref[...], axis=1, keepdims=True) + bk_ref[...]
    ef_ref[...] = f * jax.nn.sigmoid(k_log)


def _reduce_body(x_ref, x1_ref, nf_ref, Wrx_ref, Wr1n_ref, Wr2n_ref,
                 bred_ref, W2_ref, b2_ref, out_ref):
    n1 = nf_ref[:, :H]
    n2 = nf_ref[:, H:]
    n2 = jnp.where(jnp.isneginf(n2), 0.0, n2)
    new_x = (jnp.dot(x_ref[...], Wrx_ref[...], preferred_element_type=jnp.float32)
             + jnp.dot(n1, Wr1n_ref[...], preferred_element_type=jnp.float32)
             + jnp.dot(n2, Wr2n_ref[...], preferred_element_type=jnp.float32)
             + bred_ref[...])
    x2 = jnp.dot(new_x, W2_ref[...], preferred_element_type=jnp.float32) + b2_ref[...]
    out_ref[...] = x1_ref[...] + x2


def _seg_body(dst_hbm, ef_hbm, out_hbm, acc, dstbuf, dstbuf2, listbuf, rows,
              idxbuf, dlbuf, sem, sem_a, sem_b):
    wid = lax.axis_index("s") * 2 + lax.axis_index("c")
    lo = wid * NPT
    iota = lax.iota(jnp.int32, 16)
    zeros16 = jnp.zeros((16,), jnp.float32)
    neginf16 = jnp.full((16,), -jnp.inf, jnp.float32)

    def init_row(r, carry):
        for j in range(8):
            acc[r, pl.ds(16 * j, 16)] = zeros16
        for j in range(8, 16):
            acc[r, pl.ds(16 * j, 16)] = neginf16
        return carry

    lax.fori_loop(0, NPT + 1, init_row, 0)

    def process_batch(b, carry):
        base = b * GB
        for t in range(GB // 16):
            pk = listbuf[pl.ds(base + 16 * t, 16)]
            idxbuf[pl.ds(16 * t, 16)] = pk & ((1 << DL_SHIFT) - 1)
            dlbuf[pl.ds(16 * t, 16)] = lax.shift_right_logical(pk, DL_SHIFT)
        pltpu.async_copy(ef_hbm.at[idxbuf], rows, sem).wait()

        def acc_group(g, c2):
            dlv = dlbuf[pl.ds(16 * g, 16)]
            for j in range(16):
                node = dlv[j]
                i = 16 * g + j
                for t in range(8):
                    sl = pl.ds(16 * t, 16)
                    plsc.addupdate(acc.at[node, sl], rows[i, sl])
                for t in range(8, 16):
                    sl = pl.ds(16 * t, 16)
                    acc[node, sl] = jnp.maximum(acc[node, sl], rows[i, sl])
            return c2

        lax.fori_loop(0, GB // 16, acc_group, 0)
        return carry

    def drain_full(off):
        nb = off // GB
        lax.fori_loop(0, nb, process_batch, 0)
        for t in range(GB // 16):
            listbuf[pl.ds(16 * t, 16)] = listbuf[pl.ds(nb * GB + 16 * t, 16)]
        return off - nb * GB

    def scan_buf(buf, ci, off):
        def scan_vreg(v, off):
            ids = buf[pl.ds(16 * v, 16)]
            dl = ids - lo
            m = (dl >= 0) & (dl < NPT)
            cs = plsc.cumsum(m.astype(jnp.int32))
            pos = jnp.where(m, off + cs - 1, LCAP - 1)
            eids = (ci * C + 16 * v) + iota
            pk = eids | (dl << DL_SHIFT)
            plsc.store_scatter(listbuf, [pos], pk)
            return off + cs[15]

        return lax.fori_loop(0, C // 16, scan_vreg, off)

    NCHUNK = E // C

    def start_load(buf, s, ci):
        pltpu.async_copy(dst_hbm.at[pl.ds(ci * C, C)], buf, s)

    def wait_load(buf, s):
        pltpu.make_async_copy(dst_hbm.at[pl.ds(0, C)], buf, s).wait()

    start_load(dstbuf, sem_a, 0)
    start_load(dstbuf2, sem_b, 1)

    def chunk_pair(i, off):
        ca = 2 * i
        wait_load(dstbuf, sem_a)
        off = scan_buf(dstbuf, ca, off)
        start_load(dstbuf, sem_a, jnp.where(ca + 2 < NCHUNK, ca + 2, 0))
        off = drain_full(off)
        wait_load(dstbuf2, sem_b)
        off = scan_buf(dstbuf2, ca + 1, off)
        start_load(dstbuf2, sem_b, jnp.where(ca + 3 < NCHUNK, ca + 3, 0))
        off = drain_full(off)
        return off

    off = lax.fori_loop(0, NCHUNK // 2, chunk_pair, jnp.int32(0))
    wait_load(dstbuf, sem_a)
    wait_load(dstbuf2, sem_b)

    pad = jnp.full((16,), TRASH << DL_SHIFT, jnp.int32)
    def pad_tail(t, carry):
        listbuf[pl.ds(off + 16 * t, 16)] = pad
        return carry
    lax.fori_loop(0, GB // 16, pad_tail, 0)
    off_padded = ((off + GB - 1) // GB) * GB
    drain_full(off_padded)

    pltpu.sync_copy(acc.at[pl.ds(0, NPT)], out_hbm.at[pl.ds(wid * NPT, NPT)])


EPT = E // NTILES
GCH = 80


def _gather_body(src_hbm, dst_hbm, x_hbm, es_hbm, ed_hbm,
                 sidx, didx, s_a, d_a, s_b, d_b,
                 sem_sa, sem_da, sem_sb, sem_db):
    wid = lax.axis_index("s") * 2 + lax.axis_index("c")
    base = wid * EPT
    pltpu.sync_copy(src_hbm.at[pl.ds(base, EPT)], sidx)
    pltpu.sync_copy(dst_hbm.at[pl.ds(base, EPT)], didx)
    NCH = EPT // GCH

    def issue(bs, bd, s1, s2, c):
        o = c * GCH
        pltpu.async_copy(x_hbm.at[sidx.at[pl.ds(o, GCH)]], bs, s1)
        pltpu.async_copy(x_hbm.at[didx.at[pl.ds(o, GCH)]], bd, s2)

    def waitg(bs, bd, s1, s2):
        pltpu.make_async_copy(x_hbm.at[pl.ds(0, GCH)], bs, s1).wait()
        pltpu.make_async_copy(x_hbm.at[pl.ds(0, GCH)], bd, s2).wait()

    def writeout(bs, bd, c):
        o = c * GCH
        pltpu.sync_copy(bs, es_hbm.at[pl.ds(base + o, GCH)])
        pltpu.sync_copy(bd, ed_hbm.at[pl.ds(base + o, GCH)])

    issue(s_a, d_a, sem_sa, sem_da, 0)
    issue(s_b, d_b, sem_sb, sem_db, 1)

    def pair(i, carry):
        ca = 2 * i
        waitg(s_a, d_a, sem_sa, sem_da)
        writeout(s_a, d_a, ca)
        issue(s_a, d_a, sem_sa, sem_da, ca + 2)
        waitg(s_b, d_b, sem_sb, sem_db)
        writeout(s_b, d_b, ca + 1)
        issue(s_b, d_b, sem_sb, sem_db,
              jnp.where(ca + 3 < NCH, ca + 3, 0))
        return carry

    lax.fori_loop(0, (NCH - 1) // 2, pair, 0)
    waitg(s_a, d_a, sem_sa, sem_da)
    writeout(s_a, d_a, NCH - 1)
    waitg(s_b, d_b, sem_sb, sem_db)


_edge_gather = functools.partial(
    pl.kernel,
    out_type=(jax.ShapeDtypeStruct((E, H), jnp.float32),
              jax.ShapeDtypeStruct((E, H), jnp.float32)),
    mesh=plsc.VectorSubcoreMesh(core_axis_name="c", subcore_axis_name="s",
                                num_cores=2, num_subcores=16),
    scratch_types=[
        pltpu.VMEM((EPT,), jnp.int32),
        pltpu.VMEM((EPT,), jnp.int32),
        pltpu.VMEM((GCH, H), jnp.float32),
        pltpu.VMEM((GCH, H), jnp.float32),
        pltpu.VMEM((GCH, H), jnp.float32),
        pltpu.VMEM((GCH, H), jnp.float32),
        pltpu.SemaphoreType.DMA,
        pltpu.SemaphoreType.DMA,
        pltpu.SemaphoreType.DMA,
        pltpu.SemaphoreType.DMA,
    ],
    compiler_params=pltpu.CompilerParams(needs_layout_passes=False),
)(_gather_body)


_segment_reduce = functools.partial(
    pl.kernel,
    out_type=jax.ShapeDtypeStruct((NPAD, 2 * H), jnp.float32),
    mesh=plsc.VectorSubcoreMesh(core_axis_name="c", subcore_axis_name="s",
                                num_cores=2, num_subcores=16),
    scratch_types=[
        pltpu.VMEM((NPT + 1, 2 * H), jnp.float32),
        pltpu.VMEM((C,), jnp.int32),
        pltpu.VMEM((C,), jnp.int32),
        pltpu.VMEM((LCAP,), jnp.int32),
        pltpu.VMEM((GB, 2 * H), jnp.float32),
        pltpu.VMEM((GB,), jnp.int32),
        pltpu.VMEM((GB,), jnp.int32),
        pltpu.SemaphoreType.DMA,
        pltpu.SemaphoreType.DMA,
        pltpu.SemaphoreType.DMA,
    ],
    compiler_params=pltpu.CompilerParams(needs_layout_passes=False),
)(_seg_body)


def _full(shape):
    return pl.BlockSpec(shape, lambda i: (0,) * len(shape))


def kernel(nf_gc, edge_index, Wr1, br1, Wr2, br2, W1, b1, W2, b2,
           Wm1, bm1, Wm2, bm2, Wred, bred):
    n, h = nf_gc.shape
    src = edge_index[0]
    dst = edge_index[1].astype(jnp.int32)

    Wm1t = Wm1[:h]
    Wm1b = Wm1[h:]
    wk = Wm2[:, :1].T
    bk = bm2[:1].reshape(1, 1)
    Wf = Wm2[:, 1:]
    bf = bm2[1:].reshape(1, 2 * h)

    grid_n = n // NODE_BLK
    x, x1 = pl.pallas_call(
        _node_body,
        grid=(grid_n,),
        in_specs=[
            pl.BlockSpec((NODE_BLK, h), lambda i: (i, 0)),
            _full((h, h)), _full((1, h)), _full((h, h)), _full((1, h)),
            _full((h, h)), _full((1, h)),
        ],
        out_specs=[
            pl.BlockSpec((NODE_BLK, h), lambda i: (i, 0)),
            pl.BlockSpec((NODE_BLK, h), lambda i: (i, 0)),
        ],
        out_shape=[
            jax.ShapeDtypeStruct((n, h), jnp.float32),
            jax.ShapeDtypeStruct((n, h), jnp.float32),
        ],
    )(nf_gc, Wr1, br1.reshape(1, h), Wr2, br2.reshape(1, h),
      W1, b1.reshape(1, h))

    es, ed = _edge_gather(src.astype(jnp.int32), dst, x)

    grid_e = E // EDGE_BLK
    ef = pl.pallas_call(
        _edge_body,
        grid=(grid_e,),
        in_specs=[
            pl.BlockSpec((EDGE_BLK, h), lambda i: (i, 0)),
            pl.BlockSpec((EDGE_BLK, h), lambda i: (i, 0)),
            _full((h, 2 * h)), _full((h, 2 * h)),
            _full((1, 2 * h)), _full((2 * h, 2 * h)), _full((1, 2 * h)),
            _full((1, 2 * h)), _full((1, 1)),
        ],
        out_specs=pl.BlockSpec((EDGE_BLK, 2 * h), lambda i: (i, 0)),
        out_shape=jax.ShapeDtypeStruct((E, 2 * h), jnp.float32),
    )(es, ed, Wm1t.astype(jnp.bfloat16), Wm1b.astype(jnp.bfloat16),
      bm1.reshape(1, 2 * h), Wf.astype(jnp.bfloat16), bf, wk, bk)

    nfoc = _segment_reduce(dst, ef)[:n]

    out = pl.pallas_call(
        _reduce_body,
        grid=(grid_n,),
        in_specs=[
            pl.BlockSpec((NODE_BLK, h), lambda i: (i, 0)),
            pl.BlockSpec((NODE_BLK, h), lambda i: (i, 0)),
            pl.BlockSpec((NODE_BLK, 2 * h), lambda i: (i, 0)),
            _full((h, h)), _full((h, h)), _full((h, h)), _full((1, h)),
            _full((h, h)), _full((1, h)),
        ],
        out_specs=pl.BlockSpec((NODE_BLK, h), lambda i: (i, 0)),
        out_shape=jax.ShapeDtypeStruct((n, h), jnp.float32),
    )(x, x1, nfoc, Wred[:h], Wred[h:2 * h], Wred[2 * h:],
      bred.reshape(1, h), W2, b2.reshape(1, h))

    return out
